# baseline (speedup 1.0000x reference)
import functools

import jax
import jax.numpy as jnp
from jax import lax
from jax.experimental import pallas as pl
from jax.experimental.pallas import tpu as pltpu
from jax.experimental.pallas import tpu_sc as plsc

ROWS, COLS = 4096, 200
D_MODEL = 64
B = ROWS * COLS

_info = plsc.get_sparse_core_info()
_NC, _NS = _info.num_cores, _info.num_subcores
NW = _NC * _NS
BPW = B // NW
IDX_MINOR = 128
CHUNK = 512
K = CHUNK // IDX_MINOR
NCHUNK = BPW // CHUNK

_mesh = plsc.VectorSubcoreMesh(core_axis_name="c", subcore_axis_name="s")


@functools.partial(
    pl.kernel,
    mesh=_mesh,
    out_type=jax.ShapeDtypeStruct((B, D_MODEL), jnp.float32),
    compiler_params=pltpu.CompilerParams(use_tc_tiling_on_sc=False),
    scratch_types=[
        pltpu.VMEM((CHUNK,), jnp.int32),
        pltpu.VMEM((CHUNK, D_MODEL), jnp.float32),
        pltpu.SemaphoreType.DMA,
    ],
)
def _emb_lookup(idx_hbm, table_hbm, out_hbm, idx_v, rows_v, sem):
    wid = lax.axis_index("s") * _NC + lax.axis_index("c")

    def body(g, carry):
        off = wid * BPW + g * CHUNK
        pltpu.sync_copy(idx_hbm.at[pl.ds(off, CHUNK)], idx_v)
        for j in range(K):
            pltpu.async_copy(
                table_hbm.at[idx_v.at[pl.ds(j * IDX_MINOR, IDX_MINOR)]],
                rows_v.at[pl.ds(j * IDX_MINOR, IDX_MINOR)],
                sem,
            ).wait()
        pltpu.sync_copy(rows_v, out_hbm.at[pl.ds(off, CHUNK)])
        return carry

    lax.fori_loop(0, NCHUNK, body, 0)


def kernel(x, table):
    idx = x.astype(jnp.int32).reshape(B)
    out = _emb_lookup(idx, table)
    return out.reshape(ROWS, COLS, D_MODEL)


# speedup vs baseline: 1.1192x; 1.1192x over previous
import functools

import jax
import jax.numpy as jnp
from jax import lax
from jax.experimental import pallas as pl
from jax.experimental.pallas import tpu as pltpu
from jax.experimental.pallas import tpu_sc as plsc

ROWS, COLS = 4096, 200
D_MODEL = 64
B = ROWS * COLS

_info = plsc.get_sparse_core_info()
_NC, _NS = _info.num_cores, _info.num_subcores
NW = _NC * _NS
BPW = B // NW
IDX_MINOR = 128
CHUNK = 512
K = CHUNK // IDX_MINOR
NCHUNK = BPW // CHUNK
NBUF = 2

_mesh = plsc.VectorSubcoreMesh(core_axis_name="c", subcore_axis_name="s")


@functools.partial(
    pl.kernel,
    mesh=_mesh,
    out_type=jax.ShapeDtypeStruct((B, D_MODEL), jnp.float32),
    compiler_params=pltpu.CompilerParams(use_tc_tiling_on_sc=False),
    scratch_types=[
        pltpu.VMEM((BPW,), jnp.int32),
        pltpu.VMEM((CHUNK, D_MODEL), jnp.float32),
        pltpu.VMEM((CHUNK, D_MODEL), jnp.float32),
        pltpu.SemaphoreType.DMA,
        pltpu.SemaphoreType.DMA,
        pltpu.SemaphoreType.DMA,
        pltpu.SemaphoreType.DMA,
    ],
)
def _emb_lookup(idx_hbm, table_hbm, out_hbm, idx_v, rows0, rows1,
                sg0, sg1, so0, so1):
    wid = lax.axis_index("s") * _NC + lax.axis_index("c")
    base = wid * BPW
    rows = (rows0, rows1)
    sg = (sg0, sg1)
    so = (so0, so1)

    pltpu.sync_copy(idx_hbm.at[pl.ds(base, BPW)], idx_v)

    def gather_chunk(c, b):
        cps = [
            pltpu.async_copy(
                table_hbm.at[idx_v.at[pl.ds(c * CHUNK + j * IDX_MINOR,
                                            IDX_MINOR)]],
                rows[b].at[pl.ds(j * IDX_MINOR, IDX_MINOR)],
                sg[b],
            )
            for j in range(K)
        ]
        for cp in cps:
            cp.wait()

    def start_store(c, b):
        pltpu.async_copy(rows[b], out_hbm.at[pl.ds(base + c * CHUNK, CHUNK)],
                         so[b])

    def wait_store(b):
        pltpu.make_async_copy(rows[b], out_hbm.at[pl.ds(base, CHUNK)],
                              so[b]).wait()

    for b in range(NBUF):
        gather_chunk(b, b)
        start_store(b, b)

    @pl.loop(NBUF, NCHUNK, step=NBUF)
    def _ring(g):
        for b in range(NBUF):
            c = g + b
            wait_store(b)
            gather_chunk(c, b)
            start_store(c, b)

    for b in range(NBUF):
        wait_store(b)


def kernel(x, table):
    idx = x.astype(jnp.int32).reshape(B)
    out = _emb_lookup(idx, table)
    return out.reshape(ROWS, COLS, D_MODEL)
